# Initial kernel scaffold; baseline (speedup 1.0000x reference)
import jax
import jax.numpy as jnp
from jax.experimental import pallas as pl


def kernel(patches, mask_token):
    raise NotImplementedError("write your pallas kernel here")


# speedup vs baseline: 1.1099x; 1.1099x over previous
import jax
import jax.numpy as jnp
from jax.experimental import pallas as pl

MASK_PROP = 0.75


def _mask_encode_kernel(num_mask, p_ref, r_ref, m_ref, e_ref, i_ref):
    n = p_ref.shape[1]
    num_unmask = n - num_mask
    chunk = 48

    v = r_ref[0, 0, :]
    vj = v[None, :]
    jj = jax.lax.broadcasted_iota(jnp.int32, (chunk, n), 1)

    rank_chunks = []
    for c in range(0, n, chunk):
        vi = v[c : c + chunk][:, None]
        ii = jax.lax.broadcasted_iota(jnp.int32, (chunk, n), 0) + c
        cmp = (vj < vi) | ((vj == vi) & (jj < ii))
        rank_chunks.append(jnp.sum(cmp.astype(jnp.int32), axis=1))
    rank = jnp.concatenate(rank_chunks)

    kk = jax.lax.broadcasted_iota(jnp.int32, (num_unmask, n), 0) + num_mask
    onehot = (rank[None, :] == kk).astype(jnp.float32)
    e_ref[0, :num_unmask, :] = jnp.dot(
        onehot, p_ref[0], preferred_element_type=jnp.float32
    )
    e_ref[0, num_unmask:, :] = jnp.broadcast_to(
        m_ref[0, :], (num_mask, e_ref.shape[2])
    )

    col = jax.lax.broadcasted_iota(jnp.int32, (chunk, n), 1)
    for c in range(0, num_mask, chunk):
        mk = jax.lax.broadcasted_iota(jnp.int32, (chunk, n), 0) + c
        sel = rank[None, :] == mk
        i_ref[0, 0, c : c + chunk] = jnp.sum(jnp.where(sel, col, 0), axis=1)


def kernel(patches, mask_token):
    b, n, e = patches.shape
    num_mask = -(-3 * n // 4)

    rkey = jax.random.key(42)
    rand_vals = jax.random.uniform(rkey, (b, n), dtype=jnp.float32)
    rand3 = rand_vals.reshape(b, 1, n)

    import functools

    enc, idx3 = pl.pallas_call(
        functools.partial(_mask_encode_kernel, num_mask),
        grid=(b,),
        in_specs=[
            pl.BlockSpec((1, n, e), lambda i: (i, 0, 0)),
            pl.BlockSpec((1, 1, n), lambda i: (i, 0, 0)),
            pl.BlockSpec((1, e), lambda i: (0, 0)),
        ],
        out_specs=[
            pl.BlockSpec((1, n, e), lambda i: (i, 0, 0)),
            pl.BlockSpec((1, 1, num_mask), lambda i: (i, 0, 0)),
        ],
        out_shape=[
            jax.ShapeDtypeStruct((b, n, e), jnp.float32),
            jax.ShapeDtypeStruct((b, 1, num_mask), jnp.int32),
        ],
    )(patches, rand3, mask_token)
    return enc, idx3.reshape(b, num_mask)
